# baseline (speedup 1.0000x reference)
import dataclasses
import functools

import jax
import jax.numpy as jnp
from jax import lax
from jax.experimental import pallas as pl
from jax.experimental.pallas import tpu as pltpu
from jax.experimental.pallas import tpu_sc as plsc

N = 10000
NP = 10240
D = 128
OUT = 3
E = 320000

BLK = 512
GRID = NP // BLK
EC = 128
NCH = E // EC
CCH = 80
NCCH = N // CCH
TILES = 16
LANES = 16

NH = NP // 2
GB = 256
SH = NH + GB
RPS = SH // TILES
ZR = 112
WPT = NH // TILES
DRPT = NP // TILES

F32 = jnp.float32
I32 = jnp.int32


def _mesh():
    return plsc.VectorSubcoreMesh(core_axis_name="c", subcore_axis_name="s")


def _no_layout_cp():
    cp = pltpu.CompilerParams()
    if "needs_layout_passes" in pltpu.CompilerParams.__dataclass_fields__:
        cp = dataclasses.replace(cp, needs_layout_passes=False)
    return cp


def _rewrite_idx(idx_v, nbuf, cid):
    base = cid * NH
    for j in range(nbuf // LANES):
        sl = pl.ds(j * LANES, LANES)
        d = idx_v[sl]
        local = d - base
        ok = (local >= 0) & (local < NH)
        garbage = lax.iota(I32, LANES) + (NH + j * LANES % GB)
        idx_v[sl] = jnp.where(ok, local, garbage)


DGRP = 16
NDGR = 2560
NDG = NDGR // DGRP


def _deg_pair(dst_r, dst_c):
    @functools.partial(
        pl.kernel,
        out_type=[jax.ShapeDtypeStruct((NP,), F32)] * 2,
        mesh=_mesh(),
        scratch_types=[
            pltpu.VMEM((DGRP, EC), I32),
            pltpu.VMEM((EC,), F32),
            pltpu.VMEM((DRPT,), F32),
            pltpu.VMEM_SHARED((NP,), F32),
        ],
    )
    def k(dst_r_hbm, dst_c_hbm, deg_r_hbm, deg_c_hbm, idx_v, ones_v, zero_v, deg_sh):
        cid = lax.axis_index("c")
        sid = lax.axis_index("s")

        @pl.loop(0, EC // LANES)
        def _(r):
            ones_v[pl.ds(r * LANES, LANES)] = jnp.ones((LANES,), F32)

        @pl.loop(0, DRPT // LANES)
        def _(r):
            zero_v[pl.ds(r * LANES, LANES)] = jnp.zeros((LANES,), F32)

        pltpu.sync_copy(zero_v, deg_sh.at[pl.ds(sid * DRPT, DRPT)])
        plsc.subcore_barrier()

        def graph(dst_hbm):
            @pl.loop(0, NDG // TILES)
            def _(t):
                grp = sid + t * TILES
                pltpu.sync_copy(dst_hbm.at[pl.ds(grp * DGRP, DGRP), :], idx_v)
                for j in range(DGRP):
                    pltpu.sync_copy(ones_v, deg_sh.at[idx_v.at[j]], add=True)

        @pl.when(cid == 0)
        def _():
            graph(dst_r_hbm)

        @pl.when(cid == 1)
        def _():
            graph(dst_c_hbm)

        plsc.subcore_barrier()
        sl = pl.ds(sid * DRPT, DRPT)
        pltpu.sync_copy(deg_sh.at[sl], zero_v)

        @pl.when(cid == 0)
        def _():
            pltpu.sync_copy(zero_v, deg_r_hbm.at[sl])

        @pl.when(cid == 1)
        def _():
            pltpu.sync_copy(zero_v, deg_c_hbm.at[sl])

    return k(dst_r, dst_c)


REG = 160 * EC
TOT = 32 * REG
NCHT = NCH // TILES + 1
PPAIR = (NCHT + 1) // 2


def _edge_part(edge):
    @functools.partial(
        pl.kernel,
        out_type=[jax.ShapeDtypeStruct((2 * TOT,), I32),
                  jax.ShapeDtypeStruct((512,), I32)],
        mesh=_mesh(),
        compiler_params=_no_layout_cp(),
        scratch_types=[
            pltpu.VMEM((2, EC), I32),
            pltpu.VMEM((2, EC), I32),
            pltpu.VMEM((2, EC), I32),
            pltpu.VMEM((2, EC), I32),
            pltpu.VMEM((16,), I32),
            pltpu.SemaphoreType.DMA,
            pltpu.SemaphoreType.DMA,
        ],
    )
    def k(edge_hbm, part_hbm, cnt_hbm, e0_v, e1_v, p0_v, p1_v,
          cbuf_v, s0, s1):
        cid = lax.axis_index("c")
        sid = lax.axis_index("s")
        w = cid * TILES + sid
        wbase = w * REG
        nlo = cid * NH
        dump = wbase + REG - LANES + lax.iota(I32, LANES)
        bufs = ((e0_v, p0_v, s0), (e1_v, p1_v, s1))

        for e_v, pos_v, sem in bufs:
            for j in range(EC // LANES):
                sl = pl.ds(j * LANES, LANES)
                pos_v.at[0][sl] = dump
                pos_v.at[1][sl] = dump
            pltpu.async_copy(e_v.at[0], part_hbm.at[pos_v.at[0]], sem)
            pltpu.async_copy(e_v.at[1], part_hbm.at[pos_v.at[1]], sem)

        def drain(p):
            e_v, pos_v, sem = bufs[p]
            pltpu.make_async_copy(e_v.at[0], part_hbm.at[pos_v.at[0]],
                                  sem).wait()
            pltpu.make_async_copy(e_v.at[1], part_hbm.at[pos_v.at[1]],
                                  sem).wait()

        def step(p, k_id, pb):
            e_v, pos_v, sem = bufs[p]
            ch = sid + k_id * TILES
            chc = jnp.minimum(ch, NCH - 1)
            live = ch < NCH
            drain(p)
            pltpu.sync_copy(edge_hbm.at[:, pl.ds(chc * EC, EC)], e_v)
            dv = e_v.at[1]
            for j in range(EC // LANES):
                sl = pl.ds(j * LANES, LANES)
                local = dv[sl] - nlo
                mask = (local >= 0) & (local < NH) & live
                mi = mask.astype(I32)
                incl = jnp.cumsum(mi)
                pos = (wbase + pb) + (incl - mi)
                tgt = jnp.where(mask, pos, dump)
                pos_v.at[0][sl] = tgt
                pos_v.at[1][sl] = tgt + TOT
                pb = pb + jnp.sum(mi)
            pltpu.async_copy(e_v.at[0], part_hbm.at[pos_v.at[0]], sem)
            pltpu.async_copy(e_v.at[1], part_hbm.at[pos_v.at[1]], sem)
            return pb

        def body(i, pb):
            pb = step(0, 2 * i, pb)
            pb = step(1, 2 * i + 1, pb)
            return pb

        n_w = lax.fori_loop(0, PPAIR, body, jnp.int32(0))
        drain(0)
        drain(1)

        e_v, pos_v, sem = bufs[0]
        for j in range(EC // LANES):
            sl = pl.ds(j * LANES, LANES)
            e_v.at[0][sl] = jnp.zeros((LANES,), I32)
            e_v.at[1][sl] = jnp.full((LANES,), NP, I32)
            tpos = (wbase + n_w + j * LANES) + lax.iota(I32, LANES)
            pos_v.at[0][sl] = tpos
            pos_v.at[1][sl] = tpos + TOT
        pltpu.async_copy(e_v.at[0], part_hbm.at[pos_v.at[0]], sem)
        pltpu.async_copy(e_v.at[1], part_hbm.at[pos_v.at[1]], sem)
        drain(0)

        cbuf_v[...] = jnp.full((16,), n_w, I32)
        pltpu.sync_copy(cbuf_v, cnt_hbm.at[pl.ds(16 * w, 16)])

    return k(edge)


def _edge_scatter(g, part2, counts):
    @functools.partial(
        pl.kernel,
        out_type=jax.ShapeDtypeStruct((NP, D), F32),
        mesh=_mesh(),
        compiler_params=_no_layout_cp(),
        scratch_types=[
            pltpu.VMEM((2, EC), I32),
            pltpu.VMEM((2, EC), I32),
            pltpu.VMEM((EC, D), F32),
            pltpu.VMEM((EC, D), F32),
            pltpu.VMEM((ZR, D), F32),
            pltpu.VMEM_SHARED((SH, D), F32),
            pltpu.SemaphoreType.DMA,
            pltpu.SemaphoreType.DMA,
            pltpu.VMEM((512,), I32),
        ],
    )
    def k(g_hbm, part_hbm, cnt_hbm, acc_hbm, e0_v, e1_v, rows0_v, rows1_v,
          zero_v, acc_sh, sem0, sem1, cnt_v):
        cid = lax.axis_index("c")
        sid = lax.axis_index("s")
        w = cid * TILES + sid
        rbase = w * REG

        pltpu.sync_copy(cnt_hbm, cnt_v)

        @pl.loop(0, ZR)
        def _(r):
            for j in range(D // LANES):
                zero_v[r, pl.ds(LANES * j, LANES)] = jnp.zeros((LANES,), F32)

        @pl.loop(0, RPS // ZR)
        def _(b):
            pltpu.sync_copy(zero_v, acc_sh.at[pl.ds(sid * RPS + b * ZR, ZR)])

        plsc.subcore_barrier()

        n_w = jnp.max(cnt_v[pl.ds(16 * w, LANES)])
        nch = (n_w + EC - 1) // EC

        bufs = ((e0_v, rows0_v, sem0), (e1_v, rows1_v, sem1))
        base = cid * NH

        def load_idx(p, kk):
            e_v = bufs[p][0]
            pltpu.sync_copy(part_hbm.at[:, pl.ds(rbase + kk * EC, EC)], e_v)
            dv = e_v.at[1]
            for j in range(EC // LANES):
                sl = pl.ds(j * LANES, LANES)
                local = dv[sl] - base
                ok = (local >= 0) & (local < NH)
                garbage = lax.iota(I32, LANES) + (NH + j * LANES)
                dv[sl] = jnp.where(ok, local, garbage)

        def start_gather(p):
            e_v, r_v, sem = bufs[p]
            pltpu.async_copy(g_hbm.at[e_v.at[0]], r_v, sem)

        def wait_gather(p):
            e_v, r_v, sem = bufs[p]
            pltpu.make_async_copy(g_hbm.at[e_v.at[0]], r_v, sem).wait()

        def scatter(p):
            e_v, r_v, _ = bufs[p]
            pltpu.sync_copy(r_v, acc_sh.at[e_v.at[1]], add=True)

        @pl.when(nch > 0)
        def _():
            load_idx(0, 0)
            start_gather(0)

        @pl.loop(0, PPAIR)
        def _(t):
            k0 = 2 * t
            k1 = 2 * t + 1
            k2 = 2 * t + 2

            @pl.when(k1 < nch)
            def _():
                load_idx(1, k1)

            @pl.when(k0 < nch)
            def _():
                wait_gather(0)

            @pl.when(k1 < nch)
            def _():
                start_gather(1)

            @pl.when(k0 < nch)
            def _():
                scatter(0)

            @pl.when(k2 < nch)
            def _():
                load_idx(0, k2)

            @pl.when(k1 < nch)
            def _():
                wait_gather(1)

            @pl.when(k2 < nch)
            def _():
                start_gather(0)

            @pl.when(k1 < nch)
            def _():
                scatter(1)

        plsc.subcore_barrier()
        pltpu.sync_copy(acc_sh.at[pl.ds(sid * WPT, WPT)],
                        acc_hbm.at[pl.ds(cid * NH + sid * WPT, WPT)])

    return k(g, part2, counts)


def _pool_scatter(h_c, i0, i1, i2):
    @functools.partial(
        pl.kernel,
        out_type=[jax.ShapeDtypeStruct((NP, D), F32),
                  jax.ShapeDtypeStruct((NP,), F32)],
        mesh=_mesh(),
        scratch_types=[
            pltpu.VMEM((CCH,), I32),
            pltpu.VMEM((CCH, D), F32),
            pltpu.VMEM((CCH,), F32),
            pltpu.VMEM((ZR, D), F32),
            pltpu.VMEM((SH // TILES,), F32),
            pltpu.VMEM_SHARED((SH, D), F32),
            pltpu.VMEM_SHARED((SH,), F32),
        ],
    )
    def k(hc_hbm, i0_hbm, i1_hbm, i2_hbm, summ_hbm, cnt_hbm,
          idx_v, rows_v, ones_v, zero_v, zero1_v, summ_sh, cnt_sh):
        cid = lax.axis_index("c")
        sid = lax.axis_index("s")

        @pl.loop(0, ZR)
        def _(r):
            for j in range(D // LANES):
                zero_v[pl.ds(r, 1), pl.ds(LANES * j, LANES)] = (
                    jnp.zeros((1, LANES), F32))

        @pl.loop(0, CCH // LANES)
        def _(r):
            ones_v[pl.ds(r * LANES, LANES)] = jnp.ones((LANES,), F32)

        @pl.loop(0, RPS // LANES)
        def _(r):
            zero1_v[pl.ds(r * LANES, LANES)] = jnp.zeros((LANES,), F32)

        @pl.loop(0, RPS // ZR)
        def _(b):
            pltpu.sync_copy(zero_v, summ_sh.at[pl.ds(sid * RPS + b * ZR, ZR)])

        pltpu.sync_copy(zero1_v, cnt_sh.at[pl.ds(sid * RPS, RPS)])
        plsc.subcore_barrier()

        @pl.loop(0, NCCH // TILES + 1)
        def _(t):
            ch = sid + t * TILES

            @pl.when(ch < NCCH)
            def _():
                base = ch * CCH
                pltpu.sync_copy(hc_hbm.at[pl.ds(base, CCH)], rows_v)
                for ik_hbm in (i0_hbm, i1_hbm, i2_hbm):
                    pltpu.sync_copy(ik_hbm.at[pl.ds(base, CCH)], idx_v)
                    _rewrite_idx(idx_v, CCH, cid)
                    pltpu.sync_copy(rows_v, summ_sh.at[idx_v], add=True)
                    pltpu.sync_copy(ones_v, cnt_sh.at[idx_v], add=True)

        plsc.subcore_barrier()
        src_sl = pl.ds(sid * WPT, WPT)
        dst_sl = pl.ds(cid * NH + sid * WPT, WPT)
        pltpu.sync_copy(summ_sh.at[src_sl], summ_hbm.at[dst_sl])
        pltpu.sync_copy(cnt_sh.at[src_sl], zero1_v.at[pl.ds(0, WPT)])
        pltpu.sync_copy(zero1_v.at[pl.ds(0, WPT)], cnt_hbm.at[dst_sl])

    return k(h_c, i0, i1, i2)


def _dinv(deg1):
    return lax.rsqrt(deg1[:, :1] + 1.0)


_full = lambda s: pl.BlockSpec(s, lambda i: (0, 0))
_row = lambda s: pl.BlockSpec(s, lambda i: (i, 0))


def _prep(x, W1, deg):
    def body(xb, wb, db, g_o):
        g_o[...] = _dinv(db[...]) * jnp.dot(xb[...], wb[...],
                                            preferred_element_type=F32)

    return pl.pallas_call(
        body,
        grid=(GRID,),
        in_specs=[_row((BLK, D)), _full((D, D)), _row((BLK, 1))],
        out_specs=_row((BLK, D)),
        out_shape=jax.ShapeDtypeStruct((NP, D), F32),
    )(x, W1, deg)


def _mid(a, g, deg, W2, b1):
    def body(ab, gb, db, wb, bb, o):
        dinv = _dinv(db[...])
        h = jnp.maximum(dinv * (ab[...] + gb[...]) + bb[...], 0.0)
        o[...] = dinv * jnp.dot(h, wb[...], preferred_element_type=F32)

    return pl.pallas_call(
        body,
        grid=(GRID,),
        in_specs=[_row((BLK, D)), _row((BLK, D)), _row((BLK, 1)),
                  _full((D, D)), _full((1, D))],
        out_specs=_row((BLK, D)),
        out_shape=jax.ShapeDtypeStruct((NP, D), F32),
    )(a, g, deg, W2, b1)


def _finish_r(a, g, deg, b2):
    def body(ab, gb, db, bb, hr_o, sq_o):
        hr = jnp.maximum(_dinv(db[...]) * (ab[...] + gb[...]) + bb[...], 0.0)
        hr_o[...] = hr
        rowid = (pl.program_id(0) * BLK
                 + lax.broadcasted_iota(I32, (BLK, 1), 0))
        sq_o[...] = jnp.where(rowid < N,
                              -0.5 * jnp.sum(hr * hr, axis=1, keepdims=True),
                              -1e38)

    return pl.pallas_call(
        body,
        grid=(GRID,),
        in_specs=[_row((BLK, D)), _row((BLK, D)), _row((BLK, 1)),
                  _full((1, D))],
        out_specs=[_row((BLK, D)), _row((BLK, 1))],
        out_shape=[jax.ShapeDtypeStruct((NP, D), F32),
                   jax.ShapeDtypeStruct((NP, 1), F32)],
    )(a, g, deg, b2)


def _finish_c(a, g, deg, b2):
    def body(ab, gb, db, bb, hc_o):
        hc_o[...] = jnp.maximum(
            _dinv(db[...]) * (ab[...] + gb[...]) + bb[...], 0.0)

    return pl.pallas_call(
        body,
        grid=(GRID,),
        in_specs=[_row((BLK, D)), _row((BLK, D)), _row((BLK, 1)),
                  _full((1, D))],
        out_specs=_row((BLK, D)),
        out_shape=jax.ShapeDtypeStruct((NP, D), F32),
    )(a, g, deg, b2)


def _knn_top3(h_c, hrT, sq_row):
    def body(hc, hrt, sq, i0_o, i1_o, i2_o):
        s = jnp.dot(hc[...], hrt[...], preferred_element_type=F32)
        score = s + sq[...]
        iota = lax.broadcasted_iota(I32, (BLK, NP), 1)
        for j, o in enumerate((i0_o, i1_o, i2_o)):
            idx = jnp.argmax(score, axis=1).astype(I32)[:, None]
            o[...] = idx
            if j < 2:
                score = jnp.where(iota == idx, -jnp.inf, score)

    return pl.pallas_call(
        body,
        grid=(GRID,),
        in_specs=[_row((BLK, D)), _full((D, NP)), _full((1, NP))],
        out_specs=[_row((BLK, 1))] * 3,
        out_shape=[jax.ShapeDtypeStruct((NP, 1), I32)] * 3,
    )(h_c, hrT, sq_row)


def _decode(h_r, summ, cnt, W_dec, b_dec):
    def body(hr, sm, cb, w, b, o):
        pooled = sm[...] / jnp.maximum(cb[:, :1], 1.0)
        w2 = w[...]
        o[...] = (jnp.dot(hr[...], w2[:D, :], preferred_element_type=F32)
                  + jnp.dot(pooled, w2[D:, :], preferred_element_type=F32)
                  + b[...])

    return pl.pallas_call(
        body,
        grid=(GRID,),
        in_specs=[_row((BLK, D)), _row((BLK, D)), _row((BLK, 1)),
                  _full((2 * D, OUT)), _full((1, OUT))],
        out_specs=_row((BLK, OUT)),
        out_shape=jax.ShapeDtypeStruct((NP, OUT), F32),
    )(h_r, summ, cnt, W_dec, b_dec)


def kernel(x_resting, x_collider, edge_index_resting, edge_index_collider,
           W_r1, b_r1, W_r2, b_r2, W_c1, b_c1, W_c2, b_c2, W_dec, b_dec):
    pad = ((0, NP - N), (0, 0))
    x_resting = jnp.pad(x_resting, pad)
    x_collider = jnp.pad(x_collider, pad)
    dpad = (0, NDGR * EC - E)
    dst_r = jnp.pad(edge_index_resting[1], dpad,
                    constant_values=N).reshape(NDGR, EC)
    dst_c = jnp.pad(edge_index_collider[1], dpad,
                    constant_values=N).reshape(NDGR, EC)

    deg_r, deg_c = _deg_pair(dst_r, dst_c)
    deg_r = deg_r.reshape(NP, 1)
    deg_c = deg_c.reshape(NP, 1)

    p_r, c_r = _edge_part(edge_index_resting)
    p_r = p_r.reshape(2, TOT)
    p_c, c_c = _edge_part(edge_index_collider)
    p_c = p_c.reshape(2, TOT)

    g_r1 = _prep(x_resting, W_r1, deg_r)
    g_c1 = _prep(x_collider, W_c1, deg_c)
    a_r1 = _edge_scatter(g_r1, p_r, c_r)
    a_c1 = _edge_scatter(g_c1, p_c, c_c)

    g_r2 = _mid(a_r1, g_r1, deg_r, W_r2, b_r1.reshape(1, D))
    a_r2 = _edge_scatter(g_r2, p_r, c_r)
    g_c2 = _mid(a_c1, g_c1, deg_c, W_c2, b_c1.reshape(1, D))
    a_c2 = _edge_scatter(g_c2, p_c, c_c)

    h_r, sq = _finish_r(a_r2, g_r2, deg_r, b_r2.reshape(1, D))
    hrT = h_r.T
    h_c = _finish_c(a_c2, g_c2, deg_c, b_c2.reshape(1, D))

    i0, i1, i2 = _knn_top3(h_c, hrT, sq.reshape(1, NP))

    summ, cnt = _pool_scatter(h_c, i0.reshape(NP), i1.reshape(NP),
                              i2.reshape(NP))

    return _decode(h_r, summ, cnt.reshape(NP, 1), W_dec,
                   b_dec.reshape(1, OUT))[:N]


# speedup vs baseline: 7.2548x; 7.2548x over previous
import dataclasses
import functools

import jax
import jax.numpy as jnp
from jax import lax
from jax.experimental import pallas as pl
from jax.experimental.pallas import tpu as pltpu
from jax.experimental.pallas import tpu_sc as plsc

N = 10000
NP = 10240
D = 128
OUT = 3
E = 320000

BLK = 512
GRID = NP // BLK
EC = 128
NCH = E // EC
CCH = 80
NCCH = N // CCH
TILES = 16
LANES = 16

NH = NP // 2
GB = 256
SH = NH + GB
RPS = SH // TILES
ZR = 112
WPT = NH // TILES
DRPT = NP // TILES

F32 = jnp.float32
I32 = jnp.int32


def _mesh():
    return plsc.VectorSubcoreMesh(core_axis_name="c", subcore_axis_name="s")


def _no_layout_cp():
    cp = pltpu.CompilerParams()
    if "needs_layout_passes" in pltpu.CompilerParams.__dataclass_fields__:
        cp = dataclasses.replace(cp, needs_layout_passes=False)
    return cp


def _rewrite_idx(idx_v, nbuf, cid):
    base = cid * NH
    for j in range(nbuf // LANES):
        sl = pl.ds(j * LANES, LANES)
        d = idx_v[sl]
        local = d - base
        ok = (local >= 0) & (local < NH)
        garbage = lax.iota(I32, LANES) + (NH + j * LANES % GB)
        idx_v[sl] = jnp.where(ok, local, garbage)


DGRP = 16
NDGR = 2560
NDG = NDGR // DGRP


def _deg_pair(dst_r, dst_c):
    @functools.partial(
        pl.kernel,
        out_type=[jax.ShapeDtypeStruct((NP,), F32)] * 2,
        mesh=_mesh(),
        scratch_types=[
            pltpu.VMEM((DGRP, EC), I32),
            pltpu.VMEM((EC,), F32),
            pltpu.VMEM((DRPT,), F32),
            pltpu.VMEM_SHARED((NP,), F32),
        ],
    )
    def k(dst_r_hbm, dst_c_hbm, deg_r_hbm, deg_c_hbm, idx_v, ones_v, zero_v, deg_sh):
        cid = lax.axis_index("c")
        sid = lax.axis_index("s")

        @pl.loop(0, EC // LANES)
        def _(r):
            ones_v[pl.ds(r * LANES, LANES)] = jnp.ones((LANES,), F32)

        @pl.loop(0, DRPT // LANES)
        def _(r):
            zero_v[pl.ds(r * LANES, LANES)] = jnp.zeros((LANES,), F32)

        pltpu.sync_copy(zero_v, deg_sh.at[pl.ds(sid * DRPT, DRPT)])
        plsc.subcore_barrier()

        def graph(dst_hbm):
            @pl.loop(0, NDG // TILES)
            def _(t):
                grp = sid + t * TILES
                pltpu.sync_copy(dst_hbm.at[pl.ds(grp * DGRP, DGRP), :], idx_v)
                for j in range(DGRP):
                    pltpu.sync_copy(ones_v, deg_sh.at[idx_v.at[j]], add=True)

        @pl.when(cid == 0)
        def _():
            graph(dst_r_hbm)

        @pl.when(cid == 1)
        def _():
            graph(dst_c_hbm)

        plsc.subcore_barrier()
        sl = pl.ds(sid * DRPT, DRPT)
        pltpu.sync_copy(deg_sh.at[sl], zero_v)

        @pl.when(cid == 0)
        def _():
            pltpu.sync_copy(zero_v, deg_r_hbm.at[sl])

        @pl.when(cid == 1)
        def _():
            pltpu.sync_copy(zero_v, deg_c_hbm.at[sl])

    return k(dst_r, dst_c)


def _edge_scatter(g, edge):
    @functools.partial(
        pl.kernel,
        out_type=jax.ShapeDtypeStruct((NP, D), F32),
        mesh=_mesh(),
        compiler_params=_no_layout_cp(),
        scratch_types=[
            pltpu.VMEM((2, EC), I32),
            pltpu.VMEM((2, EC), I32),
            pltpu.VMEM((EC, D), F32),
            pltpu.VMEM((EC, D), F32),
            pltpu.VMEM((ZR, D), F32),
            pltpu.VMEM_SHARED((SH, D), F32),
            pltpu.SemaphoreType.DMA,
            pltpu.SemaphoreType.DMA,
        ],
    )
    def k(g_hbm, edge_hbm, acc_hbm, e0_v, e1_v, rows0_v, rows1_v,
          zero_v, acc_sh, sem0, sem1):
        cid = lax.axis_index("c")
        sid = lax.axis_index("s")

        @pl.loop(0, ZR)
        def _(r):
            for j in range(D // LANES):
                zero_v[r, pl.ds(LANES * j, LANES)] = jnp.zeros((LANES,), F32)

        @pl.loop(0, RPS // ZR)
        def _(b):
            pltpu.sync_copy(zero_v, acc_sh.at[pl.ds(sid * RPS + b * ZR, ZR)])

        plsc.subcore_barrier()

        bufs = ((e0_v, rows0_v, sem0), (e1_v, rows1_v, sem1))
        base = cid * NH

        def load_idx(p, ch):
            e_v = bufs[p][0]
            pltpu.sync_copy(edge_hbm.at[:, pl.ds(ch * EC, EC)], e_v)
            dv = e_v.at[1]
            for j in range(EC // LANES):
                sl = pl.ds(j * LANES, LANES)
                local = dv[sl] - base
                ok = (local >= 0) & (local < NH)
                garbage = lax.iota(I32, LANES) + (NH + j * LANES)
                dv[sl] = jnp.where(ok, local, garbage)

        def start_gather(p):
            e_v, r_v, sem = bufs[p]
            pltpu.async_copy(g_hbm.at[e_v.at[0]], r_v, sem)

        def wait_gather(p):
            e_v, r_v, sem = bufs[p]
            pltpu.make_async_copy(g_hbm.at[e_v.at[0]], r_v, sem).wait()

        def scatter(p):
            e_v, r_v, _ = bufs[p]
            pltpu.sync_copy(r_v, acc_sh.at[e_v.at[1]], add=True)

        load_idx(0, sid)
        start_gather(0)

        @pl.loop(0, (NCH // TILES + 2) // 2)
        def _(t):
            ch0 = sid + (2 * t) * TILES
            ch1 = sid + (2 * t + 1) * TILES
            ch2 = sid + (2 * t + 2) * TILES

            @pl.when(ch1 < NCH)
            def _():
                load_idx(1, ch1)

            @pl.when(ch0 < NCH)
            def _():
                wait_gather(0)

            @pl.when(ch1 < NCH)
            def _():
                start_gather(1)

            @pl.when(ch0 < NCH)
            def _():
                scatter(0)

            @pl.when(ch2 < NCH)
            def _():
                load_idx(0, ch2)

            @pl.when(ch1 < NCH)
            def _():
                wait_gather(1)

            @pl.when(ch2 < NCH)
            def _():
                start_gather(0)

            @pl.when(ch1 < NCH)
            def _():
                scatter(1)

        plsc.subcore_barrier()
        pltpu.sync_copy(acc_sh.at[pl.ds(sid * WPT, WPT)],
                        acc_hbm.at[pl.ds(cid * NH + sid * WPT, WPT)])

    return k(g, edge)


def _pool_scatter(h_c, i0, i1, i2):
    @functools.partial(
        pl.kernel,
        out_type=[jax.ShapeDtypeStruct((NP, D), F32),
                  jax.ShapeDtypeStruct((NP,), F32)],
        mesh=_mesh(),
        scratch_types=[
            pltpu.VMEM((CCH,), I32),
            pltpu.VMEM((CCH, D), F32),
            pltpu.VMEM((CCH,), F32),
            pltpu.VMEM((ZR, D), F32),
            pltpu.VMEM((SH // TILES,), F32),
            pltpu.VMEM_SHARED((SH, D), F32),
            pltpu.VMEM_SHARED((SH,), F32),
        ],
    )
    def k(hc_hbm, i0_hbm, i1_hbm, i2_hbm, summ_hbm, cnt_hbm,
          idx_v, rows_v, ones_v, zero_v, zero1_v, summ_sh, cnt_sh):
        cid = lax.axis_index("c")
        sid = lax.axis_index("s")

        @pl.loop(0, ZR)
        def _(r):
            for j in range(D // LANES):
                zero_v[pl.ds(r, 1), pl.ds(LANES * j, LANES)] = (
                    jnp.zeros((1, LANES), F32))

        @pl.loop(0, CCH // LANES)
        def _(r):
            ones_v[pl.ds(r * LANES, LANES)] = jnp.ones((LANES,), F32)

        @pl.loop(0, RPS // LANES)
        def _(r):
            zero1_v[pl.ds(r * LANES, LANES)] = jnp.zeros((LANES,), F32)

        @pl.loop(0, RPS // ZR)
        def _(b):
            pltpu.sync_copy(zero_v, summ_sh.at[pl.ds(sid * RPS + b * ZR, ZR)])

        pltpu.sync_copy(zero1_v, cnt_sh.at[pl.ds(sid * RPS, RPS)])
        plsc.subcore_barrier()

        @pl.loop(0, NCCH // TILES + 1)
        def _(t):
            ch = sid + t * TILES

            @pl.when(ch < NCCH)
            def _():
                base = ch * CCH
                pltpu.sync_copy(hc_hbm.at[pl.ds(base, CCH)], rows_v)
                for ik_hbm in (i0_hbm, i1_hbm, i2_hbm):
                    pltpu.sync_copy(ik_hbm.at[pl.ds(base, CCH)], idx_v)
                    _rewrite_idx(idx_v, CCH, cid)
                    pltpu.sync_copy(rows_v, summ_sh.at[idx_v], add=True)
                    pltpu.sync_copy(ones_v, cnt_sh.at[idx_v], add=True)

        plsc.subcore_barrier()
        src_sl = pl.ds(sid * WPT, WPT)
        dst_sl = pl.ds(cid * NH + sid * WPT, WPT)
        pltpu.sync_copy(summ_sh.at[src_sl], summ_hbm.at[dst_sl])
        pltpu.sync_copy(cnt_sh.at[src_sl], zero1_v.at[pl.ds(0, WPT)])
        pltpu.sync_copy(zero1_v.at[pl.ds(0, WPT)], cnt_hbm.at[dst_sl])

    return k(h_c, i0, i1, i2)


def _dinv(deg1):
    return lax.rsqrt(deg1[:, :1] + 1.0)


_full = lambda s: pl.BlockSpec(s, lambda i: (0, 0))
_row = lambda s: pl.BlockSpec(s, lambda i: (i, 0))


def _prep(x, W1, deg):
    def body(xb, wb, db, g_o):
        g_o[...] = _dinv(db[...]) * jnp.dot(xb[...], wb[...],
                                            preferred_element_type=F32)

    return pl.pallas_call(
        body,
        grid=(GRID,),
        in_specs=[_row((BLK, D)), _full((D, D)), _row((BLK, 1))],
        out_specs=_row((BLK, D)),
        out_shape=jax.ShapeDtypeStruct((NP, D), F32),
    )(x, W1, deg)


def _mid(a, g, deg, W2, b1):
    def body(ab, gb, db, wb, bb, o):
        dinv = _dinv(db[...])
        h = jnp.maximum(dinv * (ab[...] + gb[...]) + bb[...], 0.0)
        o[...] = dinv * jnp.dot(h, wb[...], preferred_element_type=F32)

    return pl.pallas_call(
        body,
        grid=(GRID,),
        in_specs=[_row((BLK, D)), _row((BLK, D)), _row((BLK, 1)),
                  _full((D, D)), _full((1, D))],
        out_specs=_row((BLK, D)),
        out_shape=jax.ShapeDtypeStruct((NP, D), F32),
    )(a, g, deg, W2, b1)


def _finish_r(a, g, deg, b2):
    def body(ab, gb, db, bb, hr_o, sq_o):
        hr = jnp.maximum(_dinv(db[...]) * (ab[...] + gb[...]) + bb[...], 0.0)
        hr_o[...] = hr
        rowid = (pl.program_id(0) * BLK
                 + lax.broadcasted_iota(I32, (BLK, 1), 0))
        sq_o[...] = jnp.where(rowid < N,
                              -0.5 * jnp.sum(hr * hr, axis=1, keepdims=True),
                              -1e38)

    return pl.pallas_call(
        body,
        grid=(GRID,),
        in_specs=[_row((BLK, D)), _row((BLK, D)), _row((BLK, 1)),
                  _full((1, D))],
        out_specs=[_row((BLK, D)), _row((BLK, 1))],
        out_shape=[jax.ShapeDtypeStruct((NP, D), F32),
                   jax.ShapeDtypeStruct((NP, 1), F32)],
    )(a, g, deg, b2)


def _finish_c(a, g, deg, b2):
    def body(ab, gb, db, bb, hc_o):
        hc_o[...] = jnp.maximum(
            _dinv(db[...]) * (ab[...] + gb[...]) + bb[...], 0.0)

    return pl.pallas_call(
        body,
        grid=(GRID,),
        in_specs=[_row((BLK, D)), _row((BLK, D)), _row((BLK, 1)),
                  _full((1, D))],
        out_specs=_row((BLK, D)),
        out_shape=jax.ShapeDtypeStruct((NP, D), F32),
    )(a, g, deg, b2)


def _knn_top3(h_c, hrT, sq_row):
    def body(hc, hrt, sq, i0_o, i1_o, i2_o):
        s = jnp.dot(hc[...], hrt[...], preferred_element_type=F32)
        score = s + sq[...]
        iota = lax.broadcasted_iota(I32, (BLK, NP), 1)
        for j, o in enumerate((i0_o, i1_o, i2_o)):
            idx = jnp.argmax(score, axis=1).astype(I32)[:, None]
            o[...] = idx
            if j < 2:
                score = jnp.where(iota == idx, -jnp.inf, score)

    return pl.pallas_call(
        body,
        grid=(GRID,),
        in_specs=[_row((BLK, D)), _full((D, NP)), _full((1, NP))],
        out_specs=[_row((BLK, 1))] * 3,
        out_shape=[jax.ShapeDtypeStruct((NP, 1), I32)] * 3,
    )(h_c, hrT, sq_row)


def _decode(h_r, summ, cnt, W_dec, b_dec):
    def body(hr, sm, cb, w, b, o):
        pooled = sm[...] / jnp.maximum(cb[:, :1], 1.0)
        w2 = w[...]
        o[...] = (jnp.dot(hr[...], w2[:D, :], preferred_element_type=F32)
                  + jnp.dot(pooled, w2[D:, :], preferred_element_type=F32)
                  + b[...])

    return pl.pallas_call(
        body,
        grid=(GRID,),
        in_specs=[_row((BLK, D)), _row((BLK, D)), _row((BLK, 1)),
                  _full((2 * D, OUT)), _full((1, OUT))],
        out_specs=_row((BLK, OUT)),
        out_shape=jax.ShapeDtypeStruct((NP, OUT), F32),
    )(h_r, summ, cnt, W_dec, b_dec)


def kernel(x_resting, x_collider, edge_index_resting, edge_index_collider,
           W_r1, b_r1, W_r2, b_r2, W_c1, b_c1, W_c2, b_c2, W_dec, b_dec):
    pad = ((0, NP - N), (0, 0))
    x_resting = jnp.pad(x_resting, pad)
    x_collider = jnp.pad(x_collider, pad)
    dpad = (0, NDGR * EC - E)
    dst_r = jnp.pad(edge_index_resting[1], dpad,
                    constant_values=N).reshape(NDGR, EC)
    dst_c = jnp.pad(edge_index_collider[1], dpad,
                    constant_values=N).reshape(NDGR, EC)

    deg_r, deg_c = _deg_pair(dst_r, dst_c)
    deg_r = deg_r.reshape(NP, 1)
    deg_c = deg_c.reshape(NP, 1)

    g_r1 = _prep(x_resting, W_r1, deg_r)
    g_c1 = _prep(x_collider, W_c1, deg_c)
    a_r1 = _edge_scatter(g_r1, edge_index_resting)
    a_c1 = _edge_scatter(g_c1, edge_index_collider)

    g_r2 = _mid(a_r1, g_r1, deg_r, W_r2, b_r1.reshape(1, D))
    a_r2 = _edge_scatter(g_r2, edge_index_resting)
    g_c2 = _mid(a_c1, g_c1, deg_c, W_c2, b_c1.reshape(1, D))
    a_c2 = _edge_scatter(g_c2, edge_index_collider)

    h_r, sq = _finish_r(a_r2, g_r2, deg_r, b_r2.reshape(1, D))
    hrT = h_r.T
    h_c = _finish_c(a_c2, g_c2, deg_c, b_c2.reshape(1, D))

    i0, i1, i2 = _knn_top3(h_c, hrT, sq.reshape(1, NP))

    summ, cnt = _pool_scatter(h_c, i0.reshape(NP), i1.reshape(NP),
                              i2.reshape(NP))

    return _decode(h_r, summ, cnt.reshape(NP, 1), W_dec,
                   b_dec.reshape(1, OUT))[:N]
